# baseline (speedup 1.0000x reference)
import jax, jax.numpy as jnp
from jax.experimental import pallas as pl

_BP = 4000
_E = 320000
_H = 256
_PREC = jax.lax.Precision.DEFAULT


def _mm_body(d, Wc, o):
    o[...] = jnp.dot(d[...], Wc[...], preferred_element_type=jnp.float32,
                     precision=_PREC)


def _mm(d, Wc):
    eb = pl.BlockSpec((_BP, _H), lambda i: (i, 0))
    return pl.pallas_call(
        _mm_body,
        grid=(_E // _BP,),
        in_specs=[eb, pl.BlockSpec((_H, _H), lambda i: (0, 0))],
        out_specs=eb,
        out_shape=jax.ShapeDtypeStruct((_E, _H), jnp.float32),
    )(d, Wc)


def kernel(x, edge_index, edge_attr, batch, atom_origin_type, is_real_bond,
           W1, b1, Wc, bc, W2, b2, Wf1, bf1, Wf2, bf2):
    N, E, B = 10000, 320000, 256
    row, col = edge_index[0], edge_index[1]
    h0 = jax.nn.relu(jnp.concatenate([x[row], edge_attr], axis=1) @ W1 + b1)
    h = h0
    rev = jnp.arange(E) ^ 1
    for l in range(4):
        a = jax.ops.segment_sum(h, col, num_segments=N)
        h = jax.nn.relu(_mm(a[row] - h[rev], Wc[l]) + bc[l] + h0)
    s = jax.ops.segment_sum(h, col, num_segments=N)
    q = jnp.concatenate([x, s], axis=1)
    hn = jax.nn.relu(q @ W2 + b2)
    pooled = jax.ops.segment_sum(hn, batch, num_segments=B)
    out = jax.nn.relu(pooled @ Wf1 + bf1) @ Wf2 + bf2
    return out.squeeze(-1)


# speedup vs baseline: 1.3511x; 1.3511x over previous
import functools
import jax
import jax.numpy as jnp
from jax import lax
from jax.experimental import pallas as pl
from jax.experimental.pallas import tpu as pltpu
from jax.experimental.pallas import tpu_sc as plsc

_N = 10000
_E = 320000
_D_FEAT = 128
_D_EDGE = 16
_HIDDEN = 256
_DEPTH = 4
_B = 256

_BP = 4000
_BN = 1000

_NW = 32
_PER_W = _E // _NW
_CH = 200


def _dot(a, b):
    return jnp.dot(a, b, preferred_element_type=jnp.float32)


def _make_sc_gather(d):
    mesh = plsc.VectorSubcoreMesh(core_axis_name="c", subcore_axis_name="s")

    @functools.partial(
        pl.kernel, mesh=mesh,
        out_type=jax.ShapeDtypeStruct((_E, d), jnp.float32),
        scratch_types=[
            pltpu.VMEM((_CH,), jnp.int32),
            pltpu.VMEM((_CH, d), jnp.float32),
            pltpu.SemaphoreType.DMA,
        ],
    )
    def gather_kernel(table_hbm, idx_hbm, out_hbm, idx_v, rows_v, sem):
        wid = lax.axis_index("s") * 2 + lax.axis_index("c")
        base = wid * _PER_W

        def chunk(j, carry):
            off = base + j * _CH
            pltpu.sync_copy(idx_hbm.at[pl.ds(off, _CH)], idx_v)
            pltpu.async_copy(table_hbm.at[idx_v], rows_v, sem).wait()
            pltpu.sync_copy(rows_v, out_hbm.at[pl.ds(off, _CH)])
            return carry

        lax.fori_loop(0, _PER_W // _CH, chunk, 0)

    return gather_kernel


_sc_gather_256 = _make_sc_gather(_HIDDEN)


def _init_body(xr, ea, W1, b1, h0_out):
    q = jnp.concatenate([xr[...], ea[...]], axis=1)
    h0_out[...] = jnp.maximum(_dot(q, W1[...]) + b1[...], 0.0)


def _edge_init(xr, ea, W1, b1):
    full = lambda shape: pl.BlockSpec(shape, lambda i: (0,) * len(shape))
    return pl.pallas_call(
        _init_body,
        grid=(_E // _BP,),
        in_specs=[pl.BlockSpec((_BP, _D_FEAT), lambda i: (i, 0)),
                  pl.BlockSpec((_BP, _D_EDGE), lambda i: (i, 0)),
                  full((_D_FEAT + _D_EDGE, _HIDDEN)), full((1, _HIDDEN))],
        out_specs=pl.BlockSpec((_BP, _HIDDEN), lambda i: (i, 0)),
        out_shape=jax.ShapeDtypeStruct((_E, _HIDDEN), jnp.float32),
    )(xr, ea, W1, b1)


def _pair_swap(hb):
    up = jnp.roll(hb, -1, axis=0)
    down = jnp.roll(hb, 1, axis=0)
    ridx = lax.broadcasted_iota(jnp.int32, hb.shape, 0)
    return jnp.where(ridx % 2 == 0, up, down)


def _layer_body(g, h, h0, Wc, bc, h_out):
    d = g[...] - _pair_swap(h[...])
    h_out[...] = jnp.maximum(_dot(d, Wc[...]) + bc[...] + h0[...], 0.0)


def _edge_layer(g, h, h0, Wc_l, bc_l):
    eb = pl.BlockSpec((_BP, _HIDDEN), lambda i: (i, 0))
    full = lambda shape: pl.BlockSpec(shape, lambda i: (0,) * len(shape))
    return pl.pallas_call(
        _layer_body,
        grid=(_E // _BP,),
        in_specs=[eb, eb, eb, full((_HIDDEN, _HIDDEN)), full((1, _HIDDEN))],
        out_specs=eb,
        out_shape=jax.ShapeDtypeStruct((_E, _HIDDEN), jnp.float32),
    )(g, h, h0, Wc_l, bc_l)


def _node_body(x, s, W2, b2, o):
    q = jnp.concatenate([x[...], s[...]], axis=1)
    o[...] = jnp.maximum(_dot(q, W2[...]) + b2[...], 0.0)


def _node_mlp(x, s, W2, b2):
    full = lambda shape: pl.BlockSpec(shape, lambda i: (0,) * len(shape))
    return pl.pallas_call(
        _node_body,
        grid=(_N // _BN,),
        in_specs=[pl.BlockSpec((_BN, _D_FEAT), lambda i: (i, 0)),
                  pl.BlockSpec((_BN, _HIDDEN), lambda i: (i, 0)),
                  full((_D_FEAT + _HIDDEN, _HIDDEN)), full((1, _HIDDEN))],
        out_specs=pl.BlockSpec((_BN, _HIDDEN), lambda i: (i, 0)),
        out_shape=jax.ShapeDtypeStruct((_N, _HIDDEN), jnp.float32),
    )(x, s, W2, b2)


def _head_body(p, Wf1, bf1, Wf2p, bf2, o):
    h = jnp.maximum(_dot(p[...], Wf1[...]) + bf1[...], 0.0)
    o[...] = _dot(h, Wf2p[...]) + bf2[...]


def _head(pooled, Wf1, bf1, Wf2, bf2):
    Wf2p = jnp.pad(Wf2, ((0, 0), (0, 127)))
    o = pl.pallas_call(
        _head_body,
        out_shape=jax.ShapeDtypeStruct((_B, 128), jnp.float32),
    )(pooled, Wf1, jnp.reshape(bf1, (1, _HIDDEN)),
      Wf2p, jnp.reshape(bf2, (1, 1)))
    return o[:, :1]


def kernel(x, edge_index, edge_attr, batch, atom_origin_type, is_real_bond,
           W1, b1, Wc, bc, W2, b2, Wf1, bf1, Wf2, bf2):
    row = edge_index[0]
    col = edge_index[1]

    xr = jnp.take(x, row, axis=0)
    h0 = _edge_init(xr, edge_attr, W1, jnp.reshape(b1, (1, _HIDDEN)))
    h = h0

    for l in range(_DEPTH):
        a = jax.ops.segment_sum(h, col, num_segments=_N)
        g = _sc_gather_256(a, row)
        h = _edge_layer(g, h, h0, Wc[l], jnp.reshape(bc[l], (1, _HIDDEN)))

    s = jax.ops.segment_sum(h, col, num_segments=_N)
    hn = _node_mlp(x, s, W2, jnp.reshape(b2, (1, _HIDDEN)))
    pooled = jax.ops.segment_sum(hn, batch, num_segments=_B)
    out = _head(pooled, Wf1, bf1, Wf2, bf2)
    return jnp.reshape(out, (_B,))


# speedup vs baseline: 1.4712x; 1.0889x over previous
import functools
import jax
import jax.numpy as jnp
from jax import lax
from jax.experimental import pallas as pl
from jax.experimental.pallas import tpu as pltpu
from jax.experimental.pallas import tpu_sc as plsc

_N = 10000
_E = 320000
_D_FEAT = 128
_D_EDGE = 16
_HIDDEN = 256
_DEPTH = 4
_B = 256

_BP = 4000
_BN = 1000

_NW = 32
_PER_W = _E // _NW
_CH = 200


def _dot(a, b):
    return jnp.dot(a, b, preferred_element_type=jnp.float32)


def _make_sc_gather(d):
    mesh = plsc.VectorSubcoreMesh(core_axis_name="c", subcore_axis_name="s")

    @functools.partial(
        pl.kernel, mesh=mesh,
        out_type=jax.ShapeDtypeStruct((_E, d), jnp.float32),
        scratch_types=[
            pltpu.VMEM((_CH,), jnp.int32),
            pltpu.VMEM((_CH, d), jnp.float32),
            pltpu.SemaphoreType.DMA,
        ],
    )
    def gather_kernel(table_hbm, idx_hbm, out_hbm, idx_v, rows_v, sem):
        wid = lax.axis_index("s") * 2 + lax.axis_index("c")
        base = wid * _PER_W

        def chunk(j, carry):
            off = base + j * _CH
            pltpu.sync_copy(idx_hbm.at[pl.ds(off, _CH)], idx_v)
            pltpu.async_copy(table_hbm.at[idx_v], rows_v, sem).wait()
            pltpu.sync_copy(rows_v, out_hbm.at[pl.ds(off, _CH)])
            return carry

        lax.fori_loop(0, _PER_W // _CH, chunk, 0)

    return gather_kernel


_sc_gather_256 = _make_sc_gather(_HIDDEN)


_SCH = 200
_RPT = 624


def _make_sc_scatter():
    mesh = plsc.VectorSubcoreMesh(core_axis_name="c", subcore_axis_name="s")

    @functools.partial(
        pl.kernel, mesh=mesh,
        out_type=jax.ShapeDtypeStruct((_N, _HIDDEN), jnp.float32),
        scratch_types=[
            pltpu.VMEM((_SCH,), jnp.int32),
            pltpu.VMEM((_SCH, 128), jnp.float32),
            pltpu.VMEM_SHARED((_N, 128), jnp.float32),
        ],
    )
    def scatter_kernel(h_hbm, col_hbm, zeros_hbm, out_hbm, idx_v, chunk_v, acc):
        c = lax.axis_index("c")
        t = lax.axis_index("s")
        rbase = t * _RPT
        pltpu.sync_copy(zeros_hbm.at[pl.ds(rbase, _RPT)],
                        acc.at[pl.ds(rbase, _RPT)])

        @pl.when(t == 15)
        def _zero_tail():
            pltpu.sync_copy(zeros_hbm.at[pl.ds(16 * _RPT, _N - 16 * _RPT)],
                            acc.at[pl.ds(16 * _RPT, _N - 16 * _RPT)])

        plsc.subcore_barrier()

        ebase = t * (_E // 16)

        def chunk(j, carry):
            off = ebase + j * _SCH
            pltpu.sync_copy(col_hbm.at[pl.ds(off, _SCH)], idx_v)
            pltpu.sync_copy(h_hbm.at[pl.ds(off, _SCH), pl.ds(c * 128, 128)],
                            chunk_v)
            pltpu.sync_copy(chunk_v, acc.at[idx_v], add=True)
            return carry

        lax.fori_loop(0, (_E // 16) // _SCH, chunk, 0)
        plsc.subcore_barrier()
        pltpu.sync_copy(acc.at[pl.ds(rbase, _RPT)],
                        out_hbm.at[pl.ds(rbase, _RPT), pl.ds(c * 128, 128)])

        @pl.when(t == 15)
        def _out_tail():
            pltpu.sync_copy(acc.at[pl.ds(16 * _RPT, _N - 16 * _RPT)],
                            out_hbm.at[pl.ds(16 * _RPT, _N - 16 * _RPT),
                                       pl.ds(c * 128, 128)])

    return scatter_kernel


_sc_scatter = _make_sc_scatter()


def _init_body(xr, ea, W1, b1, h0_out):
    q = jnp.concatenate([xr[...], ea[...]], axis=1)
    h0_out[...] = jnp.maximum(_dot(q, W1[...]) + b1[...], 0.0)


def _edge_init(xr, ea, W1, b1):
    full = lambda shape: pl.BlockSpec(shape, lambda i: (0,) * len(shape))
    return pl.pallas_call(
        _init_body,
        grid=(_E // _BP,),
        in_specs=[pl.BlockSpec((_BP, _D_FEAT), lambda i: (i, 0)),
                  pl.BlockSpec((_BP, _D_EDGE), lambda i: (i, 0)),
                  full((_D_FEAT + _D_EDGE, _HIDDEN)), full((1, _HIDDEN))],
        out_specs=pl.BlockSpec((_BP, _HIDDEN), lambda i: (i, 0)),
        out_shape=jax.ShapeDtypeStruct((_E, _HIDDEN), jnp.float32),
    )(xr, ea, W1, b1)


def _pair_swap(hb):
    up = jnp.roll(hb, -1, axis=0)
    down = jnp.roll(hb, 1, axis=0)
    ridx = lax.broadcasted_iota(jnp.int32, hb.shape, 0)
    return jnp.where(ridx % 2 == 0, up, down)


def _layer_body(g, h, h0, Wc, bc, h_out):
    d = g[...] - _pair_swap(h[...])
    h_out[...] = jnp.maximum(_dot(d, Wc[...]) + bc[...] + h0[...], 0.0)


def _edge_layer(g, h, h0, Wc_l, bc_l):
    eb = pl.BlockSpec((_BP, _HIDDEN), lambda i: (i, 0))
    full = lambda shape: pl.BlockSpec(shape, lambda i: (0,) * len(shape))
    return pl.pallas_call(
        _layer_body,
        grid=(_E // _BP,),
        in_specs=[eb, eb, eb, full((_HIDDEN, _HIDDEN)), full((1, _HIDDEN))],
        out_specs=eb,
        out_shape=jax.ShapeDtypeStruct((_E, _HIDDEN), jnp.float32),
    )(g, h, h0, Wc_l, bc_l)


def _node_body(x, s, W2, b2, o):
    q = jnp.concatenate([x[...], s[...]], axis=1)
    o[...] = jnp.maximum(_dot(q, W2[...]) + b2[...], 0.0)


def _node_mlp(x, s, W2, b2):
    full = lambda shape: pl.BlockSpec(shape, lambda i: (0,) * len(shape))
    return pl.pallas_call(
        _node_body,
        grid=(_N // _BN,),
        in_specs=[pl.BlockSpec((_BN, _D_FEAT), lambda i: (i, 0)),
                  pl.BlockSpec((_BN, _HIDDEN), lambda i: (i, 0)),
                  full((_D_FEAT + _HIDDEN, _HIDDEN)), full((1, _HIDDEN))],
        out_specs=pl.BlockSpec((_BN, _HIDDEN), lambda i: (i, 0)),
        out_shape=jax.ShapeDtypeStruct((_N, _HIDDEN), jnp.float32),
    )(x, s, W2, b2)


def _head_body(p, Wf1, bf1, Wf2p, bf2, o):
    h = jnp.maximum(_dot(p[...], Wf1[...]) + bf1[...], 0.0)
    o[...] = _dot(h, Wf2p[...]) + bf2[...]


def _head(pooled, Wf1, bf1, Wf2, bf2):
    Wf2p = jnp.pad(Wf2, ((0, 0), (0, 127)))
    o = pl.pallas_call(
        _head_body,
        out_shape=jax.ShapeDtypeStruct((_B, 128), jnp.float32),
    )(pooled, Wf1, jnp.reshape(bf1, (1, _HIDDEN)),
      Wf2p, jnp.reshape(bf2, (1, 1)))
    return o[:, :1]


def kernel(x, edge_index, edge_attr, batch, atom_origin_type, is_real_bond,
           W1, b1, Wc, bc, W2, b2, Wf1, bf1, Wf2, bf2):
    row = edge_index[0]
    col = edge_index[1]

    xr = jnp.take(x, row, axis=0)
    h0 = _edge_init(xr, edge_attr, W1, jnp.reshape(b1, (1, _HIDDEN)))
    h = h0

    for l in range(_DEPTH):
        a = jax.ops.segment_sum(h, col, num_segments=_N)
        g = _sc_gather_256(a, row)
        h = _edge_layer(g, h, h0, Wc[l], jnp.reshape(bc[l], (1, _HIDDEN)))

    s = _sc_scatter(h, col, jnp.zeros((_N, 128), jnp.float32))
    hn = _node_mlp(x, s, W2, jnp.reshape(b2, (1, _HIDDEN)))
    pooled = jax.ops.segment_sum(hn, batch, num_segments=_B)
    out = _head(pooled, Wf1, bf1, Wf2, bf2)
    return jnp.reshape(out, (_B,))
